# baseline (speedup 1.0000x reference)
import functools

import jax
import jax.numpy as jnp
from jax import lax
from jax.experimental import pallas as pl
from jax.experimental.pallas import tpu as pltpu
from jax.experimental.pallas import tpu_sc as plsc

VOCAB = 1000000
D = 64
BATCH = 4096
SEQ = 200
N = BATCH * SEQ
SCALE = 8.0

NC = 2
NS = 16
NW = NC * NS
PER_W = N // NW

IW = 128
ROWS_PER_W = PER_W // IW
K = 8
BLK = K * IW
NBLK = ROWS_PER_W // K

_mesh = plsc.VectorSubcoreMesh(core_axis_name="c", subcore_axis_name="s")


@functools.partial(
    pl.kernel,
    mesh=_mesh,
    out_type=jax.ShapeDtypeStruct((N, D), jnp.float32),
    compiler_params=pltpu.CompilerParams(use_tc_tiling_on_sc=False),
    scratch_types=[
        pltpu.VMEM((ROWS_PER_W, IW), jnp.int32),
        pltpu.VMEM((BLK, D), jnp.float32),
        pltpu.SemaphoreType.DMA,
        pltpu.SemaphoreType.DMA,
    ],
)
def _sc_embed(idx_hbm, table_hbm, out_hbm, idx_v, rows_v, isem, gsem):
    wid = lax.axis_index("s") * NC + lax.axis_index("c")
    row0 = wid * ROWS_PER_W
    pltpu.async_copy(idx_hbm.at[pl.ds(row0, ROWS_PER_W)], idx_v, isem).wait()

    @pl.loop(0, NBLK)
    def _(b):
        copies = []
        for j in range(K):
            copies.append(pltpu.async_copy(
                table_hbm.at[idx_v.at[b * K + j]],
                rows_v.at[pl.ds(j * IW, IW)],
                gsem,
            ))
        for c in copies:
            c.wait()

        @pl.loop(0, BLK)
        def _(r):
            row = rows_v.at[r]
            for c2 in range(D // 16):
                sl = pl.ds(c2 * 16, 16)
                row[sl] = row[sl] * SCALE

        pltpu.sync_copy(rows_v, out_hbm.at[pl.ds(wid * PER_W + b * BLK, BLK)])


def kernel(x, table):
    idx = x.reshape(N // IW, IW)
    out = _sc_embed(idx, table)
    return out.reshape(BATCH, SEQ, D)


# speedup vs baseline: 1.0994x; 1.0994x over previous
import functools

import jax
import jax.numpy as jnp
from jax import lax
from jax.experimental import pallas as pl
from jax.experimental.pallas import tpu as pltpu
from jax.experimental.pallas import tpu_sc as plsc

VOCAB = 1000000
D = 64
BATCH = 4096
SEQ = 200
N = BATCH * SEQ
SCALE = 8.0

NC = 2
NS = 16
NW = NC * NS
PER_W = N // NW

IW = 128
ROWS_PER_W = PER_W // IW
K = 4
BLK = K * IW
NBLK = ROWS_PER_W // K

_mesh = plsc.VectorSubcoreMesh(core_axis_name="c", subcore_axis_name="s")


@functools.partial(
    pl.kernel,
    mesh=_mesh,
    out_type=jax.ShapeDtypeStruct((N, D), jnp.float32),
    compiler_params=pltpu.CompilerParams(use_tc_tiling_on_sc=False),
    scratch_types=[
        pltpu.VMEM((ROWS_PER_W, IW), jnp.int32),
        pltpu.VMEM((BLK, D), jnp.float32),
        pltpu.VMEM((BLK, D), jnp.float32),
        pltpu.SemaphoreType.DMA,
        pltpu.SemaphoreType.DMA,
        pltpu.SemaphoreType.DMA,
    ],
)
def _sc_embed(idx_hbm, table_hbm, out_hbm, idx_v, rows_a, rows_b, isem,
              gsem_a, gsem_b):
    wid = lax.axis_index("s") * NC + lax.axis_index("c")
    row0 = wid * ROWS_PER_W
    pltpu.async_copy(idx_hbm.at[pl.ds(row0, ROWS_PER_W)], idx_v, isem).wait()

    def fire(b, buf, sem):
        for j in range(K):
            pltpu.async_copy(
                table_hbm.at[idx_v.at[b * K + j]],
                buf.at[pl.ds(j * IW, IW)],
                sem,
            )

    def drain(b, buf, sem):
        for j in range(K):
            pltpu.make_async_copy(
                table_hbm.at[idx_v.at[b * K + j]],
                buf.at[pl.ds(j * IW, IW)],
                sem,
            ).wait()

    def scale(buf):
        @pl.loop(0, BLK, step=4)
        def _(r):
            for rr in range(4):
                row = buf.at[r + rr]
                for c2 in range(D // 16):
                    sl = pl.ds(c2 * 16, 16)
                    row[sl] = row[sl] * SCALE

    def store(b, buf):
        pltpu.sync_copy(buf, out_hbm.at[pl.ds(wid * PER_W + b * BLK, BLK)])

    fire(0, rows_a, gsem_a)

    @pl.loop(0, NBLK // 2)
    def _(t):
        b0 = 2 * t
        fire(b0 + 1, rows_b, gsem_b)
        drain(b0, rows_a, gsem_a)
        scale(rows_a)
        store(b0, rows_a)

        @pl.when(b0 + 2 < NBLK)
        def _():
            fire(b0 + 2, rows_a, gsem_a)

        drain(b0 + 1, rows_b, gsem_b)
        scale(rows_b)
        store(b0 + 1, rows_b)


def kernel(x, table):
    idx = x.reshape(N // IW, IW)
    out = _sc_embed(idx, table)
    return out.reshape(BATCH, SEQ, D)
